# baseline (speedup 1.0000x reference)
import jax
import jax.numpy as jnp
from jax import lax
from jax.experimental import pallas as pl
from jax.experimental.pallas import tpu as pltpu
from jax.experimental.pallas import tpu_sc as plsc

_GRID = 16
_STRIDE = 16
_BATCH = 16384
_NS, _L = 16, 16
_BPW = _BATCH // _NS
_TAB = _GRID * _GRID


def _spline_body(a_hbm, b_hbm, c0_hbm, c1_hbm, c2_hbm, out_hbm,
                 a_v, b_v, c0_v, c1_v, c2_v, out_v, sem):
    off = lax.axis_index("s") * _BPW
    copies = [
        pltpu.async_copy(a_hbm.at[pl.ds(off, _BPW)], a_v, sem),
        pltpu.async_copy(b_hbm.at[pl.ds(off, _BPW)], b_v, sem),
        pltpu.async_copy(c0_hbm, c0_v, sem),
        pltpu.async_copy(c1_hbm, c1_v, sem),
        pltpu.async_copy(c2_hbm, c2_v, sem),
    ]
    for c in copies:
        c.wait()
    @pl.loop(0, _BPW // _L, unroll=2)
    def _iter(j):
        av = a_v[pl.ds(j * _L, _L)]
        bv = b_v[pl.ds(j * _L, _L)]
        idx = (av & (_GRID * _STRIDE - _STRIDE)) | lax.shift_right_logical(bv, 4)
        offa = (av & (_STRIDE - 1)).astype(jnp.float32)
        offb = (bv & (_STRIDE - 1)).astype(jnp.float32)
        c0 = plsc.load_gather(c0_v, [idx])
        c1 = plsc.load_gather(c1_v, [idx])
        c2 = plsc.load_gather(c2_v, [idx])
        out_v[pl.ds(j * _L, _L)] = c0 + c1 * offa + c2 * offb
    pltpu.sync_copy(out_v, out_hbm.at[pl.ds(off, _BPW)])


def kernel(a, b, coeffs):
    cf = coeffs.reshape(_TAB, 3)
    run = pl.kernel(
        _spline_body,
        out_type=jax.ShapeDtypeStruct((_BATCH,), jnp.float32),
        mesh=plsc.VectorSubcoreMesh(core_axis_name="c", subcore_axis_name="s",
                                    num_cores=1),
        compiler_params=pltpu.CompilerParams(
            needs_layout_passes=False,
            disable_bounds_checks=True,
            disable_semaphore_checks=True,
            skip_device_barrier=True,
        ),
        scratch_types=[
            pltpu.VMEM((_BPW,), jnp.int32),
            pltpu.VMEM((_BPW,), jnp.int32),
            pltpu.VMEM((_TAB,), jnp.float32),
            pltpu.VMEM((_TAB,), jnp.float32),
            pltpu.VMEM((_TAB,), jnp.float32),
            pltpu.VMEM((_BPW,), jnp.float32),
            pltpu.SemaphoreType.DMA,
        ],
    )
    out = run(a, b, cf[:, 0], cf[:, 1], cf[:, 2])
    return out.reshape(_BATCH, 1)


# speedup vs baseline: 1.0336x; 1.0336x over previous
import jax
import jax.numpy as jnp
from jax import lax
from jax.experimental import pallas as pl
from jax.experimental.pallas import tpu as pltpu
from jax.experimental.pallas import tpu_sc as plsc

_GRID = 16
_STRIDE = 16
_BATCH = 16384
_NS, _L = 16, 16
_BPW = _BATCH // _NS
_TAB = _GRID * _GRID * 3


def _spline_body(a_hbm, b_hbm, tab_hbm, out_hbm, a_v, b_v, tab_v, out_v, sem):
    off = lax.axis_index("s") * _BPW
    copies = [
        pltpu.async_copy(a_hbm.at[pl.ds(off, _BPW)], a_v, sem),
        pltpu.async_copy(b_hbm.at[pl.ds(off, _BPW)], b_v, sem),
        pltpu.async_copy(tab_hbm, tab_v, sem),
    ]
    for c in copies:
        c.wait()

    @pl.loop(0, _BPW // _L, unroll=1)
    def _iter(j):
        av = a_v[pl.ds(j * _L, _L)]
        bv = b_v[pl.ds(j * _L, _L)]
        idx = (av & (_GRID * _STRIDE - _STRIDE)) | lax.shift_right_logical(bv, 4)
        idx3 = idx * 3
        offa = (av & (_STRIDE - 1)).astype(jnp.float32)
        offb = (bv & (_STRIDE - 1)).astype(jnp.float32)
        c0 = plsc.load_gather(tab_v, [idx3])
        c1 = plsc.load_gather(tab_v, [idx3 + 1])
        c2 = plsc.load_gather(tab_v, [idx3 + 2])
        out_v[pl.ds(j * _L, _L)] = c0 + c1 * offa + c2 * offb

    pltpu.sync_copy(out_v, out_hbm.at[pl.ds(off, _BPW)])


def kernel(a, b, coeffs):
    run = pl.kernel(
        _spline_body,
        out_type=jax.ShapeDtypeStruct((_BATCH,), jnp.float32),
        mesh=plsc.VectorSubcoreMesh(core_axis_name="c", subcore_axis_name="s",
                                    num_cores=1),
        compiler_params=pltpu.CompilerParams(
            needs_layout_passes=False,
            disable_bounds_checks=True,
            disable_semaphore_checks=True,
            skip_device_barrier=True,
        ),
        scratch_types=[
            pltpu.VMEM((_BPW,), jnp.int32),
            pltpu.VMEM((_BPW,), jnp.int32),
            pltpu.VMEM((_TAB,), jnp.float32),
            pltpu.VMEM((_BPW,), jnp.float32),
            pltpu.SemaphoreType.DMA,
        ],
    )
    out = run(a, b, coeffs.reshape(_TAB))
    return out.reshape(_BATCH, 1)
